# Initial kernel scaffold; baseline (speedup 1.0000x reference)
import jax
import jax.numpy as jnp
from jax.experimental import pallas as pl


def kernel(x0, emb_table, W0, W1, neighbors):
    raise NotImplementedError("write your pallas kernel here")


# speedup vs baseline: 5.8946x; 5.8946x over previous
import functools

import jax
import jax.numpy as jnp
from jax.experimental import pallas as pl

N = 50000
D = 256
S = 6
BN = 1000


def _project_tables_kernel(emb_ref, w0_ref, w1_ref, t0_ref, t1_ref):
    emb = emb_ref[...]
    scale = 1.0 / S
    for w_ref, t_ref in ((w0_ref, t0_ref), (w1_ref, t1_ref)):
        wb = w_ref[:, D:]
        t = jax.lax.dot_general(
            emb, wb, (((1,), (1,)), ((), ())),
            preferred_element_type=jnp.float32,
            precision=jax.lax.Precision.HIGHEST,
        )
        t_ref[...] = t * scale


def _sage_kernel(x0_ref, nb_ref, w0_ref, w1_ref, t0_ref, t1_ref, out_ref):
    nb = nb_ref[...]
    col_ids = jax.lax.broadcasted_iota(jnp.int32, (BN, D), 1)
    counts = jnp.zeros((BN, D), jnp.float32)
    for s in range(S):
        counts = counts + (nb[:, s][:, None] == col_ids).astype(jnp.float32)

    emb = x0_ref[...]
    for w_ref, t_ref in ((w0_ref, t0_ref), (w1_ref, t1_ref)):
        wa = w_ref[:, :D]
        h = jax.lax.dot_general(
            emb, wa, (((1,), (1,)), ((), ())),
            preferred_element_type=jnp.float32,
            precision=jax.lax.Precision.HIGHEST,
        )
        h = h + jax.lax.dot_general(
            counts, t_ref[...], (((1,), (0,)), ((), ())),
            preferred_element_type=jnp.float32,
            precision=jax.lax.Precision.HIGHEST,
        )
        emb = jnp.maximum(h, 0.0)
    out_ref[...] = emb


@jax.jit
def kernel(x0, emb_table, W0, W1, neighbors):
    nb = neighbors.astype(jnp.int32)
    t0, t1 = pl.pallas_call(
        _project_tables_kernel,
        out_shape=(
            jax.ShapeDtypeStruct((D, D), jnp.float32),
            jax.ShapeDtypeStruct((D, D), jnp.float32),
        ),
    )(emb_table, W0, W1)

    grid = N // BN
    full = pl.BlockSpec((D, 2 * D), lambda i: (0, 0))
    small = pl.BlockSpec((D, D), lambda i: (0, 0))
    out = pl.pallas_call(
        _sage_kernel,
        grid=(grid,),
        in_specs=[
            pl.BlockSpec((BN, D), lambda i: (i, 0)),
            pl.BlockSpec((BN, S), lambda i: (i, 0)),
            full, full, small, small,
        ],
        out_specs=pl.BlockSpec((BN, D), lambda i: (i, 0)),
        out_shape=jax.ShapeDtypeStruct((N, D), jnp.float32),
    )(x0, nb, W0, W1, t0, t1)
    return out


# speedup vs baseline: 17.2492x; 2.9263x over previous
import functools

import jax
import jax.numpy as jnp
from jax.experimental import pallas as pl

N = 50000
D = 256
S = 6
BN = 1000


def _project_tables_kernel(emb_ref, w0_ref, w1_ref, t0_ref, t1_ref):
    emb = emb_ref[...]
    scale = 1.0 / S
    for w_ref, t_ref in ((w0_ref, t0_ref), (w1_ref, t1_ref)):
        wb = w_ref[:, D:]
        t = jax.lax.dot_general(
            emb, wb, (((1,), (1,)), ((), ())),
            preferred_element_type=jnp.float32,
            precision=jax.lax.Precision.HIGHEST,
        )
        t_ref[...] = t * scale


def _sage_kernel(x0_ref, nb_ref, w0_ref, w1_ref, t0_ref, t1_ref, out_ref):
    nb = nb_ref[...]
    col_ids = jax.lax.broadcasted_iota(jnp.int32, (BN, D), 1)
    counts = jnp.zeros((BN, D), jnp.float32)
    for s in range(S):
        counts = counts + (nb[:, s][:, None] == col_ids).astype(jnp.float32)

    emb = x0_ref[...]
    for w_ref, t_ref in ((w0_ref, t0_ref), (w1_ref, t1_ref)):
        wa = w_ref[:, :D]
        h = jax.lax.dot_general(
            emb, wa, (((1,), (1,)), ((), ())),
            preferred_element_type=jnp.float32,
        )
        h = h + jax.lax.dot_general(
            counts, t_ref[...], (((1,), (0,)), ((), ())),
            preferred_element_type=jnp.float32,
        )
        emb = jnp.maximum(h, 0.0)
    out_ref[...] = emb


@jax.jit
def kernel(x0, emb_table, W0, W1, neighbors):
    nb = neighbors.astype(jnp.int32)
    t0, t1 = pl.pallas_call(
        _project_tables_kernel,
        out_shape=(
            jax.ShapeDtypeStruct((D, D), jnp.float32),
            jax.ShapeDtypeStruct((D, D), jnp.float32),
        ),
    )(emb_table, W0, W1)

    grid = N // BN
    full = pl.BlockSpec((D, 2 * D), lambda i: (0, 0))
    small = pl.BlockSpec((D, D), lambda i: (0, 0))
    out = pl.pallas_call(
        _sage_kernel,
        grid=(grid,),
        in_specs=[
            pl.BlockSpec((BN, D), lambda i: (i, 0)),
            pl.BlockSpec((BN, S), lambda i: (i, 0)),
            full, full, small, small,
        ],
        out_specs=pl.BlockSpec((BN, D), lambda i: (i, 0)),
        out_shape=jax.ShapeDtypeStruct((N, D), jnp.float32),
    )(x0, nb, W0, W1, t0, t1)
    return out
